# Initial kernel scaffold; baseline (speedup 1.0000x reference)
import functools

import jax
import jax.numpy as jnp
from jax import lax
from jax.experimental import pallas as pl
from jax.experimental.pallas import tpu as pltpu
from jax.experimental.pallas import tpu_sc as plsc

NC = 2
NS = 16

_mesh = lambda: plsc.VectorSubcoreMesh(core_axis_name="c", subcore_axis_name="s")


def _pick_chunk(ept: int) -> int:
    for c in (16000, 10000, 8000, 5000, 4000, 2000, 1000, 800, 400, 200, 104, 8):
        if ept % c == 0 and c % 8 == 0:
            return c
    raise ValueError(f"edges-per-tile {ept} not chunkable")


@functools.partial(jax.jit, static_argnames=("n", "e"))
def _sc_degree(dst, ones_c, zero_n, *, n, e):
    epc = e // NC
    ept = epc // NS
    c = _pick_chunk(ept)
    nch = ept // c

    @functools.partial(
        pl.kernel,
        out_type=jax.ShapeDtypeStruct((NC, n), jnp.float32),
        mesh=_mesh(),
        scratch_types=[
            pltpu.VMEM((c,), jnp.int32),
            pltpu.VMEM((c,), jnp.float32),
            pltpu.VMEM_SHARED((n,), jnp.float32),
        ],
    )
    def deg_kernel(dst_hbm, ones_hbm, zero_hbm, out_hbm, idx_v, ones_v, deg_sh):
        cid = lax.axis_index("c")
        sid = lax.axis_index("s")

        @pl.when(sid == 0)
        def _():
            pltpu.sync_copy(zero_hbm, deg_sh)

        pltpu.sync_copy(ones_hbm, ones_v)
        plsc.subcore_barrier()
        base = cid * epc + sid * ept
        for i in range(nch):
            pltpu.sync_copy(dst_hbm.at[pl.ds(base + i * c, c)], idx_v)
            pltpu.sync_copy(ones_v, deg_sh.at[idx_v], add=True)
        plsc.subcore_barrier()

        @pl.when(sid == 0)
        def _():
            pltpu.sync_copy(deg_sh, out_hbm.at[cid])

    return deg_kernel(dst, ones_c, zero_n)


@functools.partial(jax.jit, static_argnames=("n", "e", "k"))
def _sc_accumulate(src, dst, tables, zero_n, *, n, e, k):
    epc = e // NC
    ept = epc // NS
    c = _pick_chunk(ept)
    nch = ept // c

    @functools.partial(
        pl.kernel,
        out_type=[jax.ShapeDtypeStruct((NC, n), jnp.float32)] * k,
        mesh=_mesh(),
        scratch_types=[
            pltpu.VMEM((c,), jnp.int32),
            pltpu.VMEM((c,), jnp.int32),
        ]
        + [pltpu.VMEM((c,), jnp.float32) for _ in range(k)]
        + [pltpu.VMEM_SHARED((n,), jnp.float32) for _ in range(k)]
        + [pltpu.SemaphoreType.DMA],
    )
    def acc_kernel(src_hbm, dst_hbm, *rest):
        tabs = rest[:k]
        zero_hbm = rest[k]
        outs = rest[k + 1 : 2 * k + 1]
        sidx_v = rest[2 * k + 1]
        didx_v = rest[2 * k + 2]
        gs = rest[2 * k + 3 : 3 * k + 3]
        accs = rest[3 * k + 3 : 4 * k + 3]
        sem = rest[4 * k + 3]

        cid = lax.axis_index("c")
        sid = lax.axis_index("s")

        for j in range(k):
            @pl.when(sid == j)
            def _(j=j):
                pltpu.sync_copy(zero_hbm, accs[j])

        plsc.subcore_barrier()
        base = cid * epc + sid * ept
        for i in range(nch):
            pltpu.sync_copy(src_hbm.at[pl.ds(base + i * c, c)], sidx_v)
            pltpu.sync_copy(dst_hbm.at[pl.ds(base + i * c, c)], didx_v)
            for j in range(k):
                pltpu.async_copy(tabs[j].at[sidx_v], gs[j], sem).wait()
                pltpu.sync_copy(gs[j], accs[j].at[didx_v], add=True)
        plsc.subcore_barrier()

        for j in range(k):
            @pl.when(sid == j)
            def _(j=j):
                pltpu.sync_copy(accs[j], outs[j].at[cid])

    outs = acc_kernel(src, dst, *tables, zero_n)
    return outs if k > 1 else (outs,)


def _vspec(bn):
    return pl.BlockSpec((bn,), lambda j: (j,))


@functools.partial(jax.jit, static_argnames=("n",))
def _tc_prep(deg0, deg1, x0, x1, *, n):
    bn = 2048

    def body(d0, d1, x0r, x1r, dis_o, u0_o, u1_o):
        deg = d0[...] + d1[...] + 1.0
        dis = lax.rsqrt(deg)
        dis_o[...] = dis
        u0_o[...] = dis * x0r[...]
        u1_o[...] = dis * x1r[...]

    return pl.pallas_call(
        body,
        grid=(pl.cdiv(n, bn),),
        in_specs=[_vspec(bn)] * 4,
        out_specs=[_vspec(bn)] * 3,
        out_shape=[jax.ShapeDtypeStruct((n,), jnp.float32)] * 3,
    )(deg0, deg1, x0, x1)


@functools.partial(jax.jit, static_argnames=("n",))
def _tc_mid(a00, a10, u0, a01, a11, u1, dis, w1t, b1c, w2, *, n):
    bn = 1024
    hid = w1t.shape[0]

    def body(a00r, a10r, u0r, a01r, a11r, u1r, disr, w1r, b1r, w2r, w_o):
        dis_v = disr[...]
        pre0 = dis_v * (a00r[...] + a10r[...] + u0r[...])
        pre1 = dis_v * (a01r[...] + a11r[...] + u1r[...])
        w1 = w1r[...]
        h = w1[:, 0:1] * pre0[None, :] + w1[:, 1:2] * pre1[None, :] + b1r[...]
        h = jnp.maximum(h, 0.0)
        z = jnp.sum(h * w2r[...], axis=0)
        w_o[...] = dis_v * z

    wspec = lambda s: pl.BlockSpec(s, lambda j: (0, 0))
    return pl.pallas_call(
        body,
        grid=(pl.cdiv(n, bn),),
        in_specs=[_vspec(bn)] * 7 + [wspec((hid, 2)), wspec((hid, 1)), wspec((hid, 1))],
        out_specs=_vspec(bn),
        out_shape=jax.ShapeDtypeStruct((n,), jnp.float32),
    )(a00, a10, u0, a01, a11, u1, dis, w1t, b1c, w2)


@functools.partial(jax.jit, static_argnames=("n",))
def _tc_fin(a2a, a2b, w, dis, b2c, *, n):
    bn = 2048

    def body(ar, br, wr, disr, b2r, out_o):
        v = disr[...] * (ar[...] + br[...] + wr[...])
        out_o[...] = v[:, None] + b2r[...]

    return pl.pallas_call(
        body,
        grid=(pl.cdiv(n, bn),),
        in_specs=[_vspec(bn)] * 4 + [pl.BlockSpec((1, 1), lambda j: (0, 0))],
        out_specs=pl.BlockSpec((bn, 1), lambda j: (j, 0)),
        out_shape=jax.ShapeDtypeStruct((n, 1), jnp.float32),
    )(a2a, a2b, w, dis, b2c)


def kernel(x, edge_index, W1, b1, W2, b2):
    n = x.shape[0]
    e = edge_index.shape[1]
    hid = W1.shape[1]

    ei = edge_index.astype(jnp.int32)
    src = ei[0]
    dst = ei[1]
    x0 = x[:, 0]
    x1 = x[:, 1]
    zero_n = jnp.zeros((n,), jnp.float32)
    epc = e // NC
    c = _pick_chunk(epc // NS)
    ones_c = jnp.ones((c,), jnp.float32)

    degp = _sc_degree(dst, ones_c, zero_n, n=n, e=e)
    dis, u0, u1 = _tc_prep(degp[0], degp[1], x0, x1, n=n)
    acc0, acc1 = _sc_accumulate(src, dst, (u0, u1), zero_n, n=n, e=e, k=2)
    w = _tc_mid(acc0[0], acc0[1], u0, acc1[0], acc1[1], u1, dis,
                W1.T, b1.reshape(hid, 1), W2, n=n)
    (acc2,) = _sc_accumulate(src, dst, (w,), zero_n, n=n, e=e, k=1)
    return _tc_fin(acc2[0], acc2[1], w, dis, b2.reshape(1, 1), n=n)


# speedup vs baseline: 122.7530x; 122.7530x over previous
import functools

import jax
import jax.numpy as jnp
from jax import lax
from jax.experimental import pallas as pl
from jax.experimental.pallas import tpu as pltpu
from jax.experimental.pallas import tpu_sc as plsc

NC = 2
NS = 16

_mesh = lambda: plsc.VectorSubcoreMesh(core_axis_name="c", subcore_axis_name="s")


def _pick_chunk(ept: int) -> int:
    for c in (16000, 10000, 8000, 5000, 4000, 2000, 1000, 800, 400, 200, 104, 8):
        if ept % c == 0 and c % 8 == 0:
            return c
    raise ValueError(f"edges-per-tile {ept} not chunkable")


@functools.partial(jax.jit, static_argnames=("n", "e"))
def _sc_degree(dst, ones_c, zero_n, *, n, e):
    epc = e // NC
    ept = epc // NS
    c = _pick_chunk(ept)
    nch = ept // c

    @functools.partial(
        pl.kernel,
        out_type=jax.ShapeDtypeStruct((NC, n), jnp.float32),
        mesh=_mesh(),
        scratch_types=[
            pltpu.VMEM((c,), jnp.int32),
            pltpu.VMEM((c,), jnp.float32),
            pltpu.VMEM_SHARED((n,), jnp.float32),
        ],
    )
    def deg_kernel(dst_hbm, ones_hbm, zero_hbm, out_hbm, idx_v, ones_v, deg_sh):
        cid = lax.axis_index("c")
        sid = lax.axis_index("s")

        @pl.when(sid == 0)
        def _():
            pltpu.sync_copy(zero_hbm, deg_sh)

        pltpu.sync_copy(ones_hbm, ones_v)
        plsc.subcore_barrier()
        base = cid * epc + sid * ept
        for i in range(nch):
            pltpu.sync_copy(dst_hbm.at[pl.ds(base + i * c, c)], idx_v)
            pltpu.sync_copy(ones_v, deg_sh.at[idx_v], add=True)
        plsc.subcore_barrier()

        @pl.when(sid == 0)
        def _():
            pltpu.sync_copy(deg_sh, out_hbm.at[cid])

    return deg_kernel(dst, ones_c, zero_n)


@functools.partial(jax.jit, static_argnames=("n", "e", "k"))
def _sc_accumulate(src, dst, tables, zero_n, *, n, e, k):
    epc = e // NC
    ept = epc // NS
    c = _pick_chunk(ept)
    nch = ept // c

    @functools.partial(
        pl.kernel,
        out_type=[jax.ShapeDtypeStruct((NC, n), jnp.float32)] * k,
        mesh=_mesh(),
        scratch_types=[
            pltpu.VMEM((c,), jnp.int32),
            pltpu.VMEM((c,), jnp.int32),
        ]
        + [pltpu.VMEM((c,), jnp.float32) for _ in range(k)]
        + [pltpu.VMEM_SHARED((n,), jnp.float32) for _ in range(k)]
        + [pltpu.SemaphoreType.DMA],
    )
    def acc_kernel(src_hbm, dst_hbm, *rest):
        tabs = rest[:k]
        zero_hbm = rest[k]
        outs = rest[k + 1 : 2 * k + 1]
        sidx_v = rest[2 * k + 1]
        didx_v = rest[2 * k + 2]
        gs = rest[2 * k + 3 : 3 * k + 3]
        accs = rest[3 * k + 3 : 4 * k + 3]
        sem = rest[4 * k + 3]

        cid = lax.axis_index("c")
        sid = lax.axis_index("s")

        for j in range(k):
            @pl.when(sid == j)
            def _(j=j):
                pltpu.sync_copy(zero_hbm, accs[j])

        plsc.subcore_barrier()
        base = cid * epc + sid * ept
        for i in range(nch):
            pltpu.sync_copy(src_hbm.at[pl.ds(base + i * c, c)], sidx_v)
            pltpu.sync_copy(dst_hbm.at[pl.ds(base + i * c, c)], didx_v)
            for j in range(k):
                pltpu.async_copy(tabs[j].at[sidx_v], gs[j], sem).wait()
                pltpu.sync_copy(gs[j], accs[j].at[didx_v], add=True)
        plsc.subcore_barrier()

        for j in range(k):
            @pl.when(sid == j)
            def _(j=j):
                pltpu.sync_copy(accs[j], outs[j].at[cid])

    outs = acc_kernel(src, dst, *tables, zero_n)
    if not isinstance(outs, (list, tuple)):
        outs = (outs,)
    return tuple(outs)


def _vspec(bn):
    return pl.BlockSpec((bn,), lambda j: (j,))


@functools.partial(jax.jit, static_argnames=("n",))
def _tc_prep(deg0, deg1, x0, x1, *, n):
    bn = 2048

    def body(d0, d1, x0r, x1r, dis_o, u0_o, u1_o):
        deg = d0[...] + d1[...] + 1.0
        dis = lax.rsqrt(deg)
        dis_o[...] = dis
        u0_o[...] = dis * x0r[...]
        u1_o[...] = dis * x1r[...]

    return pl.pallas_call(
        body,
        grid=(pl.cdiv(n, bn),),
        in_specs=[_vspec(bn)] * 4,
        out_specs=[_vspec(bn)] * 3,
        out_shape=[jax.ShapeDtypeStruct((n,), jnp.float32)] * 3,
    )(deg0, deg1, x0, x1)


@functools.partial(jax.jit, static_argnames=("n",))
def _tc_mid(a00, a10, u0, a01, a11, u1, dis, w1t, b1c, w2, *, n):
    bn = 1024
    hid = w1t.shape[0]

    def body(a00r, a10r, u0r, a01r, a11r, u1r, disr, w1r, b1r, w2r, w_o):
        dis_v = disr[...]
        pre0 = dis_v * (a00r[...] + a10r[...] + u0r[...])
        pre1 = dis_v * (a01r[...] + a11r[...] + u1r[...])
        w1 = w1r[...]
        h = w1[:, 0:1] * pre0[None, :] + w1[:, 1:2] * pre1[None, :] + b1r[...]
        h = jnp.maximum(h, 0.0)
        z = jnp.sum(h * w2r[...], axis=0)
        w_o[...] = dis_v * z

    wspec = lambda s: pl.BlockSpec(s, lambda j: (0, 0))
    return pl.pallas_call(
        body,
        grid=(pl.cdiv(n, bn),),
        in_specs=[_vspec(bn)] * 7 + [wspec((hid, 2)), wspec((hid, 1)), wspec((hid, 1))],
        out_specs=_vspec(bn),
        out_shape=jax.ShapeDtypeStruct((n,), jnp.float32),
    )(a00, a10, u0, a01, a11, u1, dis, w1t, b1c, w2)


@functools.partial(jax.jit, static_argnames=("n",))
def _tc_fin(a2a, a2b, w, dis, b2c, *, n):
    bn = 2048

    def body(ar, br, wr, disr, b2r, out_o):
        v = disr[...] * (ar[...] + br[...] + wr[...])
        out_o[...] = v[:, None] + b2r[...]

    return pl.pallas_call(
        body,
        grid=(pl.cdiv(n, bn),),
        in_specs=[_vspec(bn)] * 4 + [pl.BlockSpec((1, 1), lambda j: (0, 0))],
        out_specs=pl.BlockSpec((bn, 1), lambda j: (j, 0)),
        out_shape=jax.ShapeDtypeStruct((n, 1), jnp.float32),
    )(a2a, a2b, w, dis, b2c)


def kernel(x, edge_index, W1, b1, W2, b2):
    n = x.shape[0]
    e = edge_index.shape[1]
    hid = W1.shape[1]

    ei = edge_index.astype(jnp.int32)
    src = ei[0]
    dst = ei[1]
    x0 = x[:, 0]
    x1 = x[:, 1]
    zero_n = jnp.zeros((n,), jnp.float32)
    epc = e // NC
    c = _pick_chunk(epc // NS)
    ones_c = jnp.ones((c,), jnp.float32)

    degp = _sc_degree(dst, ones_c, zero_n, n=n, e=e)
    dis, u0, u1 = _tc_prep(degp[0], degp[1], x0, x1, n=n)
    acc0, acc1 = _sc_accumulate(src, dst, (u0, u1), zero_n, n=n, e=e, k=2)
    w = _tc_mid(acc0[0], acc0[1], u0, acc1[0], acc1[1], u1, dis,
                W1.T, b1.reshape(hid, 1), W2, n=n)
    (acc2,) = _sc_accumulate(src, dst, (w,), zero_n, n=n, e=e, k=1)
    return _tc_fin(acc2[0], acc2[1], w, dis, b2.reshape(1, 1), n=n)


# speedup vs baseline: 204.1750x; 1.6633x over previous
import functools

import jax
import jax.numpy as jnp
from jax import lax
from jax.experimental import pallas as pl
from jax.experimental.pallas import tpu as pltpu
from jax.experimental.pallas import tpu_sc as plsc

NC = 2
NS = 16

_mesh = lambda: plsc.VectorSubcoreMesh(core_axis_name="c", subcore_axis_name="s")


def _pick_chunk(ept: int) -> int:
    for c in (16000, 10000, 8000, 5000, 4000, 2000, 1000, 800, 400, 200, 104, 8):
        if ept % c == 0 and c % 8 == 0:
            return c
    raise ValueError(f"edges-per-tile {ept} not chunkable")


def _pick_chunk_tab(ept: int, n: int, k: int) -> int:
    cmax = min((131071 - 128) // 6,
               (2_000_000 - 2 * k * (n + 128)) // (6 * NS))
    best = 0
    for d in range(1, ept // 16 + 1):
        c = 16 * d
        if c <= cmax and ept % c == 0:
            best = c
    if best == 0:
        raise ValueError(f"edges-per-tile {ept} not chunkable")
    return best


@functools.partial(jax.jit, static_argnames=("n", "e"))
def _sc_degree(dst, ones_c, zero_n, *, n, e):
    epc = e // NC
    ept = epc // NS
    c = _pick_chunk(ept)
    nch = ept // c

    @functools.partial(
        pl.kernel,
        out_type=jax.ShapeDtypeStruct((NC, n), jnp.float32),
        mesh=_mesh(),
        scratch_types=[
            pltpu.VMEM((c,), jnp.int32),
            pltpu.VMEM((c,), jnp.float32),
            pltpu.VMEM_SHARED((n,), jnp.float32),
        ],
    )
    def deg_kernel(dst_hbm, ones_hbm, zero_hbm, out_hbm, idx_v, ones_v, deg_sh):
        cid = lax.axis_index("c")
        sid = lax.axis_index("s")

        @pl.when(sid == 0)
        def _():
            pltpu.sync_copy(zero_hbm, deg_sh)

        pltpu.sync_copy(ones_hbm, ones_v)
        plsc.subcore_barrier()
        base = cid * epc + sid * ept
        for i in range(nch):
            pltpu.sync_copy(dst_hbm.at[pl.ds(base + i * c, c)], idx_v)
            pltpu.sync_copy(ones_v, deg_sh.at[idx_v], add=True)
        plsc.subcore_barrier()

        @pl.when(sid == 0)
        def _():
            pltpu.sync_copy(deg_sh, out_hbm.at[cid])

    return deg_kernel(dst, ones_c, zero_n)


@functools.partial(jax.jit, static_argnames=("n", "e", "k"))
def _sc_accumulate(src, dst, tables, zero_n, *, n, e, k):
    epc = e // NC
    gt = NS // k
    ept = epc // gt
    c = _pick_chunk_tab(ept, n, k)
    nch = ept // c

    @functools.partial(
        pl.kernel,
        out_type=[jax.ShapeDtypeStruct((NC, n), jnp.float32)] * k,
        mesh=_mesh(),
        scratch_types=[pltpu.VMEM((c,), jnp.int32) for _ in range(4)]
        + [pltpu.VMEM((c,), jnp.float32) for _ in range(2)]
        + [pltpu.VMEM_SHARED((n,), jnp.float32) for _ in range(2 * k)]
        + [pltpu.SemaphoreType.DMA for _ in range(6)],
    )
    def acc_kernel(src_hbm, dst_hbm, *rest):
        tabs = rest[:k]
        zero_hbm = rest[k]
        outs = rest[k + 1 : 2 * k + 1]
        r = rest[2 * k + 1 :]
        sidx = [r[0], r[1]]
        didx = [r[2], r[3]]
        gbuf = [r[4], r[5]]
        tab_sh = r[6 : 6 + k]
        accs = r[6 + k : 6 + 2 * k]
        sem_i = [r[6 + 2 * k], r[7 + 2 * k]]
        sem_g = [r[8 + 2 * k], r[9 + 2 * k]]
        sem_s = [r[10 + 2 * k], r[11 + 2 * k]]

        cid = lax.axis_index("c")
        sid = lax.axis_index("s")

        for g in range(k):
            @pl.when(sid == 2 * g)
            def _(g=g):
                pltpu.sync_copy(tabs[g], tab_sh[g])

            @pl.when(sid == 2 * g + 1)
            def _(g=g):
                pltpu.sync_copy(zero_hbm, accs[g])

        plsc.subcore_barrier()
        base = cid * epc + (sid // k) * ept

        for g in range(k):
            @pl.when(sid % k == g)
            def _(g=g):
                def fire_idx(i, b):
                    bb = base + i * c
                    d1 = pltpu.async_copy(src_hbm.at[pl.ds(bb, c)], sidx[b], sem_i[b])
                    d2 = pltpu.async_copy(dst_hbm.at[pl.ds(bb, c)], didx[b], sem_i[b])
                    return (d1, d2)

                descs_i = [None, None]
                descs_s = [None, None]
                descs_i[0] = fire_idx(0, 0)
                for i in range(nch):
                    cur = i % 2
                    oth = 1 - cur
                    for d in descs_i[cur]:
                        d.wait()
                    if descs_s[cur] is not None:
                        descs_s[cur].wait()
                        descs_s[cur] = None
                    gd = pltpu.async_copy(tab_sh[g].at[sidx[cur]], gbuf[cur],
                                          sem_g[cur])
                    gd.wait()
                    descs_s[cur] = pltpu.async_copy(
                        gbuf[cur], accs[g].at[didx[cur]], sem_s[cur], add=True)
                    if i + 1 < nch:
                        if descs_s[oth] is not None:
                            descs_s[oth].wait()
                            descs_s[oth] = None
                        descs_i[oth] = fire_idx(i + 1, oth)
                for dsc in descs_s:
                    if dsc is not None:
                        dsc.wait()

        plsc.subcore_barrier()

        for j in range(k):
            @pl.when(sid == j)
            def _(j=j):
                pltpu.sync_copy(accs[j], outs[j].at[cid])

    outs = acc_kernel(src, dst, *tables, zero_n)
    if not isinstance(outs, (list, tuple)):
        outs = (outs,)
    return tuple(outs)


def _vspec(bn):
    return pl.BlockSpec((bn,), lambda j: (j,))


@functools.partial(jax.jit, static_argnames=("n",))
def _tc_prep(deg0, deg1, x0, x1, *, n):
    bn = 2048

    def body(d0, d1, x0r, x1r, dis_o, u0_o, u1_o):
        deg = d0[...] + d1[...] + 1.0
        dis = lax.rsqrt(deg)
        dis_o[...] = dis
        u0_o[...] = dis * x0r[...]
        u1_o[...] = dis * x1r[...]

    return pl.pallas_call(
        body,
        grid=(pl.cdiv(n, bn),),
        in_specs=[_vspec(bn)] * 4,
        out_specs=[_vspec(bn)] * 3,
        out_shape=[jax.ShapeDtypeStruct((n,), jnp.float32)] * 3,
    )(deg0, deg1, x0, x1)


@functools.partial(jax.jit, static_argnames=("n",))
def _tc_mid(a00, a10, u0, a01, a11, u1, dis, w1t, b1c, w2, *, n):
    bn = 1024
    hid = w1t.shape[0]

    def body(a00r, a10r, u0r, a01r, a11r, u1r, disr, w1r, b1r, w2r, w_o):
        dis_v = disr[...]
        pre0 = dis_v * (a00r[...] + a10r[...] + u0r[...])
        pre1 = dis_v * (a01r[...] + a11r[...] + u1r[...])
        w1 = w1r[...]
        h = w1[:, 0:1] * pre0[None, :] + w1[:, 1:2] * pre1[None, :] + b1r[...]
        h = jnp.maximum(h, 0.0)
        z = jnp.sum(h * w2r[...], axis=0)
        w_o[...] = dis_v * z

    wspec = lambda s: pl.BlockSpec(s, lambda j: (0, 0))
    return pl.pallas_call(
        body,
        grid=(pl.cdiv(n, bn),),
        in_specs=[_vspec(bn)] * 7 + [wspec((hid, 2)), wspec((hid, 1)), wspec((hid, 1))],
        out_specs=_vspec(bn),
        out_shape=jax.ShapeDtypeStruct((n,), jnp.float32),
    )(a00, a10, u0, a01, a11, u1, dis, w1t, b1c, w2)


@functools.partial(jax.jit, static_argnames=("n",))
def _tc_fin(a2a, a2b, w, dis, b2c, *, n):
    bn = 2048

    def body(ar, br, wr, disr, b2r, out_o):
        v = disr[...] * (ar[...] + br[...] + wr[...])
        out_o[...] = v[:, None] + b2r[...]

    return pl.pallas_call(
        body,
        grid=(pl.cdiv(n, bn),),
        in_specs=[_vspec(bn)] * 4 + [pl.BlockSpec((1, 1), lambda j: (0, 0))],
        out_specs=pl.BlockSpec((bn, 1), lambda j: (j, 0)),
        out_shape=jax.ShapeDtypeStruct((n, 1), jnp.float32),
    )(a2a, a2b, w, dis, b2c)


def kernel(x, edge_index, W1, b1, W2, b2):
    n = x.shape[0]
    e = edge_index.shape[1]
    hid = W1.shape[1]

    ei = edge_index.astype(jnp.int32)
    src = ei[0]
    dst = ei[1]
    x0 = x[:, 0]
    x1 = x[:, 1]
    zero_n = jnp.zeros((n,), jnp.float32)
    epc = e // NC
    c = _pick_chunk(epc // NS)
    ones_c = jnp.ones((c,), jnp.float32)

    degp = _sc_degree(dst, ones_c, zero_n, n=n, e=e)
    dis, u0, u1 = _tc_prep(degp[0], degp[1], x0, x1, n=n)
    acc0, acc1 = _sc_accumulate(src, dst, (u0, u1), zero_n, n=n, e=e, k=2)
    w = _tc_mid(acc0[0], acc0[1], u0, acc1[0], acc1[1], u1, dis,
                W1.T, b1.reshape(hid, 1), W2, n=n)
    (acc2,) = _sc_accumulate(src, dst, (w,), zero_n, n=n, e=e, k=1)
    return _tc_fin(acc2[0], acc2[1], w, dis, b2.reshape(1, 1), n=n)


# speedup vs baseline: 214.6620x; 1.0514x over previous
import functools

import jax
import jax.numpy as jnp
from jax import lax
from jax.experimental import pallas as pl
from jax.experimental.pallas import tpu as pltpu
from jax.experimental.pallas import tpu_sc as plsc

NC = 2
NS = 16

_mesh = lambda: plsc.VectorSubcoreMesh(core_axis_name="c", subcore_axis_name="s")


def _pick_chunk(ept: int) -> int:
    for c in (16000, 10000, 8000, 5000, 4000, 2000, 1000, 800, 400, 200, 104, 8):
        if ept % c == 0 and c % 8 == 0:
            return c
    raise ValueError(f"edges-per-tile {ept} not chunkable")


def _pick_chunk_tab(ept: int, n: int, k: int) -> int:
    cmax = min((131071 - 128) // (4 + 2 * k),
               (1_900_000 - 2 * k * (n + 256)) // ((4 + 2 * k) * NS))
    best = 0
    for d in range(1, ept // 16 + 1):
        c = 16 * d
        if c <= cmax and ept % c == 0:
            best = c
    if best == 0:
        raise ValueError(f"edges-per-tile {ept} not chunkable")
    return best


@functools.partial(jax.jit, static_argnames=("n", "e"))
def _sc_degree(dst, ones_c, zero_n, *, n, e):
    epc = e // NC
    ept = epc // NS
    c = _pick_chunk(ept)
    nch = ept // c

    @functools.partial(
        pl.kernel,
        out_type=jax.ShapeDtypeStruct((NC, n), jnp.float32),
        mesh=_mesh(),
        scratch_types=[
            pltpu.VMEM((c,), jnp.int32),
            pltpu.VMEM((c,), jnp.float32),
            pltpu.VMEM_SHARED((n,), jnp.float32),
        ],
    )
    def deg_kernel(dst_hbm, ones_hbm, zero_hbm, out_hbm, idx_v, ones_v, deg_sh):
        cid = lax.axis_index("c")
        sid = lax.axis_index("s")

        @pl.when(sid == 0)
        def _():
            pltpu.sync_copy(zero_hbm, deg_sh)

        pltpu.sync_copy(ones_hbm, ones_v)
        plsc.subcore_barrier()
        base = cid * epc + sid * ept
        for i in range(nch):
            pltpu.sync_copy(dst_hbm.at[pl.ds(base + i * c, c)], idx_v)
            pltpu.sync_copy(ones_v, deg_sh.at[idx_v], add=True)
        plsc.subcore_barrier()

        @pl.when(sid == 0)
        def _():
            pltpu.sync_copy(deg_sh, out_hbm.at[cid])

    return deg_kernel(dst, ones_c, zero_n)


@functools.partial(jax.jit, static_argnames=("n", "e", "k"))
def _sc_accumulate(src, dst, tables, zero_n, *, n, e, k):
    epc = e // NC
    ept = epc // NS
    c = _pick_chunk_tab(ept, n, k)
    nch = ept // c

    @functools.partial(
        pl.kernel,
        out_type=[jax.ShapeDtypeStruct((NC, n), jnp.float32)] * k,
        mesh=_mesh(),
        scratch_types=[pltpu.VMEM((c,), jnp.int32) for _ in range(4)]
        + [pltpu.VMEM((c,), jnp.float32) for _ in range(2 * k)]
        + [pltpu.VMEM_SHARED((n,), jnp.float32) for _ in range(2 * k)]
        + [pltpu.SemaphoreType.DMA for _ in range(6)],
    )
    def acc_kernel(src_hbm, dst_hbm, *rest):
        tabs = rest[:k]
        zero_hbm = rest[k]
        outs = rest[k + 1 : 2 * k + 1]
        r = rest[2 * k + 1 :]
        sidx = [r[0], r[1]]
        didx = [r[2], r[3]]
        gbuf = [[r[4 + 2 * t], r[5 + 2 * t]] for t in range(k)]
        tab_sh = r[4 + 2 * k : 4 + 3 * k]
        accs = r[4 + 3 * k : 4 + 4 * k]
        sem_i = [r[4 + 4 * k], r[5 + 4 * k]]
        sem_g = [r[6 + 4 * k], r[7 + 4 * k]]
        sem_s = [r[8 + 4 * k], r[9 + 4 * k]]

        cid = lax.axis_index("c")
        sid = lax.axis_index("s")

        for g in range(k):
            @pl.when(sid == 2 * g)
            def _(g=g):
                pltpu.sync_copy(tabs[g], tab_sh[g])

            @pl.when(sid == 2 * g + 1)
            def _(g=g):
                pltpu.sync_copy(zero_hbm, accs[g])

        plsc.subcore_barrier()
        base = cid * epc + sid * ept

        def fire_idx(i, b):
            bb = base + i * c
            d1 = pltpu.async_copy(src_hbm.at[pl.ds(bb, c)], sidx[b], sem_i[b])
            d2 = pltpu.async_copy(dst_hbm.at[pl.ds(bb, c)], didx[b], sem_i[b])
            return (d1, d2)

        descs_i = [None, None]
        descs_s = [None, None]
        descs_i[0] = fire_idx(0, 0)
        for i in range(nch):
            cur = i % 2
            oth = 1 - cur
            for d in descs_i[cur]:
                d.wait()
            if descs_s[cur] is not None:
                for d in descs_s[cur]:
                    d.wait()
                descs_s[cur] = None
            gds = [pltpu.async_copy(tab_sh[t].at[sidx[cur]], gbuf[t][cur],
                                    sem_g[cur]) for t in range(k)]
            for gd in gds:
                gd.wait()
            descs_s[cur] = [pltpu.async_copy(
                gbuf[t][cur], accs[t].at[didx[cur]], sem_s[cur], add=True)
                for t in range(k)]
            if i + 1 < nch:
                if descs_s[oth] is not None:
                    for d in descs_s[oth]:
                        d.wait()
                    descs_s[oth] = None
                descs_i[oth] = fire_idx(i + 1, oth)
        for dsc in descs_s:
            if dsc is not None:
                for d in dsc:
                    d.wait()

        plsc.subcore_barrier()

        for j in range(k):
            @pl.when(sid == 2 * k + j)
            def _(j=j):
                pltpu.sync_copy(accs[j], outs[j].at[cid])

    outs = acc_kernel(src, dst, *tables, zero_n)
    if not isinstance(outs, (list, tuple)):
        outs = (outs,)
    return tuple(outs)


def _vspec(bn):
    return pl.BlockSpec((bn,), lambda j: (j,))


@functools.partial(jax.jit, static_argnames=("n",))
def _tc_prep(deg0, deg1, x0, x1, *, n):
    bn = 2048

    def body(d0, d1, x0r, x1r, dis_o, u0_o, u1_o):
        deg = d0[...] + d1[...] + 1.0
        dis = lax.rsqrt(deg)
        dis_o[...] = dis
        u0_o[...] = dis * x0r[...]
        u1_o[...] = dis * x1r[...]

    return pl.pallas_call(
        body,
        grid=(pl.cdiv(n, bn),),
        in_specs=[_vspec(bn)] * 4,
        out_specs=[_vspec(bn)] * 3,
        out_shape=[jax.ShapeDtypeStruct((n,), jnp.float32)] * 3,
    )(deg0, deg1, x0, x1)


@functools.partial(jax.jit, static_argnames=("n",))
def _tc_mid(a00, a10, u0, a01, a11, u1, dis, w1t, b1c, w2, *, n):
    bn = 1024
    hid = w1t.shape[0]

    def body(a00r, a10r, u0r, a01r, a11r, u1r, disr, w1r, b1r, w2r, w_o):
        dis_v = disr[...]
        pre0 = dis_v * (a00r[...] + a10r[...] + u0r[...])
        pre1 = dis_v * (a01r[...] + a11r[...] + u1r[...])
        w1 = w1r[...]
        h = w1[:, 0:1] * pre0[None, :] + w1[:, 1:2] * pre1[None, :] + b1r[...]
        h = jnp.maximum(h, 0.0)
        z = jnp.sum(h * w2r[...], axis=0)
        w_o[...] = dis_v * z

    wspec = lambda s: pl.BlockSpec(s, lambda j: (0, 0))
    return pl.pallas_call(
        body,
        grid=(pl.cdiv(n, bn),),
        in_specs=[_vspec(bn)] * 7 + [wspec((hid, 2)), wspec((hid, 1)), wspec((hid, 1))],
        out_specs=_vspec(bn),
        out_shape=jax.ShapeDtypeStruct((n,), jnp.float32),
    )(a00, a10, u0, a01, a11, u1, dis, w1t, b1c, w2)


@functools.partial(jax.jit, static_argnames=("n",))
def _tc_fin(a2a, a2b, w, dis, b2c, *, n):
    bn = 2048

    def body(ar, br, wr, disr, b2r, out_o):
        v = disr[...] * (ar[...] + br[...] + wr[...])
        out_o[...] = v[:, None] + b2r[...]

    return pl.pallas_call(
        body,
        grid=(pl.cdiv(n, bn),),
        in_specs=[_vspec(bn)] * 4 + [pl.BlockSpec((1, 1), lambda j: (0, 0))],
        out_specs=pl.BlockSpec((bn, 1), lambda j: (j, 0)),
        out_shape=jax.ShapeDtypeStruct((n, 1), jnp.float32),
    )(a2a, a2b, w, dis, b2c)


@jax.jit
def _run(x, edge_index, W1, b1, W2, b2):
    n = x.shape[0]
    e = edge_index.shape[1]
    hid = W1.shape[1]

    ei = edge_index.astype(jnp.int32)
    src = ei[0]
    dst = ei[1]
    x0 = x[:, 0]
    x1 = x[:, 1]
    zero_n = jnp.zeros((n,), jnp.float32)
    epc = e // NC
    c = _pick_chunk(epc // NS)
    ones_c = jnp.ones((c,), jnp.float32)

    degp = _sc_degree(dst, ones_c, zero_n, n=n, e=e)
    dis, u0, u1 = _tc_prep(degp[0], degp[1], x0, x1, n=n)
    acc0, acc1 = _sc_accumulate(src, dst, (u0, u1), zero_n, n=n, e=e, k=2)
    w = _tc_mid(acc0[0], acc0[1], u0, acc1[0], acc1[1], u1, dis,
                W1.T, b1.reshape(hid, 1), W2, n=n)
    (acc2,) = _sc_accumulate(src, dst, (w,), zero_n, n=n, e=e, k=1)
    return _tc_fin(acc2[0], acc2[1], w, dis, b2.reshape(1, 1), n=n)


def kernel(x, edge_index, W1, b1, W2, b2):
    return _run(x, edge_index, W1, b1, W2, b2)
